# baseline (speedup 1.0000x reference)
import functools

import jax
import jax.numpy as jnp
import numpy as np
from jax import lax
from jax.experimental import pallas as pl
from jax.experimental.pallas import tpu as pltpu
from jax.experimental.pallas import tpu_sc as plsc

B, N, C = 4, 8192, 128
NPOINT, NSAMPLE = 1024, 32
D1, D2_, D3 = 128, 128, 256
R2 = float(np.float32(0.4) * np.float32(0.4))

NC, NS, L = 2, 16, 16
NW = NC * NS
NGRP = N // L
ROWS = B * NPOINT * NSAMPLE

BPH = 2
SPQ = BPH * NPOINT
SPW = SPQ // NW
ROWS_H = SPQ * NSAMPLE

_mesh = plsc.VectorSubcoreMesh(core_axis_name="c", subcore_axis_name="s",
                               num_cores=NC, num_subcores=NS)
_sc_params = pltpu.CompilerParams(needs_layout_passes=False)


def _wid():
    return lax.axis_index("s") * NC + lax.axis_index("c")


def _rbf16(v):
    u = plsc.bitcast(v, jnp.int32)
    lsb = jnp.bitwise_and(lax.shift_right_logical(u, 16), 1)
    r = jnp.bitwise_and(u + 0x7FFF + lsb, jnp.int32(-65536))
    return plsc.bitcast(r, jnp.float32)


def _make_ballquery(h):
  @functools.partial(
      pl.kernel,
      out_type=[
          jax.ShapeDtypeStruct((SPQ * 3,), jnp.float32),
          jax.ShapeDtypeStruct((ROWS_H,), jnp.int32),
          jax.ShapeDtypeStruct((ROWS_H,), jnp.int32),
          jax.ShapeDtypeStruct((3 * ROWS_H,), jnp.float32),
      ],
      mesh=_mesh,
      compiler_params=_sc_params,
      scratch_types=[
          pltpu.VMEM((N,), jnp.float32),
          pltpu.VMEM((N,), jnp.float32),
          pltpu.VMEM((N,), jnp.float32),
          pltpu.VMEM((N,), jnp.float32),
          pltpu.VMEM((N,), jnp.float32),
          pltpu.VMEM((N,), jnp.float32),
          pltpu.VMEM((N,), jnp.float32),
          pltpu.VMEM((SPW,), jnp.int32),
          pltpu.VMEM((SPW,), jnp.float32),
          pltpu.VMEM((SPW,), jnp.float32),
          pltpu.VMEM((SPW,), jnp.float32),
          pltpu.VMEM((SPW * 3,), jnp.float32),
          pltpu.VMEM((48,), jnp.int32),
          pltpu.VMEM((SPW * NSAMPLE,), jnp.int32),
          pltpu.VMEM((SPW * NSAMPLE,), jnp.int32),
          pltpu.VMEM((SPW * NSAMPLE,), jnp.float32),
          pltpu.VMEM((SPW * NSAMPLE,), jnp.float32),
          pltpu.VMEM((SPW * NSAMPLE,), jnp.float32),
      ],
  )
  def _ballquery(xyzT_hbm, inds_hbm, nx_hbm, idx_hbm, gidx_hbm, dxyz_hbm,
                 px, py, pz, pxr, pyr, pzr, x2a, ind_v, cxc, cyc, czc,
                 nx_tile, idxbuf, idx_tile, gidx_tile, dxt, dyt, dzt):
    wid = _wid()
    wpb = NW // BPH
    b = h * BPH + wid // wpb

    pltpu.sync_copy(xyzT_hbm.at[pl.ds(0 * B * N + b * N, N)], px)
    pltpu.sync_copy(xyzT_hbm.at[pl.ds(1 * B * N + b * N, N)], py)
    pltpu.sync_copy(xyzT_hbm.at[pl.ds(2 * B * N + b * N, N)], pz)
    pltpu.sync_copy(inds_hbm.at[pl.ds(wid * SPW, SPW)], ind_v)

    lane = lax.iota(jnp.int32, L)

    def round_grp(g, carry):
        xs = px[pl.ds(g * L, L)]
        ys = py[pl.ds(g * L, L)]
        zs = pz[pl.ds(g * L, L)]
        pxr[pl.ds(g * L, L)] = _rbf16(xs)
        pyr[pl.ds(g * L, L)] = _rbf16(ys)
        pzr[pl.ds(g * L, L)] = _rbf16(zs)
        x2a[pl.ds(g * L, L)] = (xs * xs + ys * ys) + zs * zs
        return carry

    lax.fori_loop(0, NGRP, round_grp, jnp.int32(0))

    for g in range(SPW // L):
        iv = ind_v[pl.ds(g * L, L)]
        cxg = plsc.load_gather(px, [iv])
        cyg = plsc.load_gather(py, [iv])
        czg = plsc.load_gather(pz, [iv])
        cxc[pl.ds(g * L, L)] = cxg
        cyc[pl.ds(g * L, L)] = cyg
        czc[pl.ds(g * L, L)] = czg
        tri = 3 * (lane + g * L)
        plsc.store_scatter(nx_tile, [tri], cxg)
        plsc.store_scatter(nx_tile, [tri + 1], cyg)
        plsc.store_scatter(nx_tile, [tri + 2], czg)
    pltpu.sync_copy(nx_tile, nx_hbm.at[pl.ds(wid * SPW * 3, SPW * 3)])
    zeros16 = jnp.zeros((L,), jnp.int32)
    ones16 = jnp.ones((L,), jnp.int32)
    bN = b * N

    def per_centroid(j, carry):
        jv = jnp.full((L,), j, jnp.int32)
        cxb = plsc.load_gather(cxc, [jv])
        cyb = plsc.load_gather(cyc, [jv])
        czb = plsc.load_gather(czc, [jv])
        q2 = (cxb * cxb + cyb * cyb) + czb * czb
        cxh, cyh, czh = _rbf16(cxb), _rbf16(cyb), _rbf16(czb)

        def cond(st):
            g, cnt = st
            return jnp.logical_and(g < NGRP, cnt < NSAMPLE)

        def body(st):
            g, cnt = st
            dot = (cxh * pxr[pl.ds(g * L, L)] + cyh * pyr[pl.ds(g * L, L)]) \
                + czh * pzr[pl.ds(g * L, L)]
            d2 = (q2 + x2a[pl.ds(g * L, L)]) - 2.0 * dot
            m = d2 < R2
            plsc.store_compressed(idxbuf.at[pl.ds(cnt, L)], lane + g * L, mask=m)
            cnt = cnt + jnp.sum(jnp.where(m, ones16, zeros16))
            return g + 1, cnt

        _, cnt = lax.while_loop(cond, body,
                                (jnp.int32(0), jnp.int32(0)))

        v0 = idxbuf[pl.ds(0, L)]
        v1 = idxbuf[pl.ds(L, L)]
        cntv = jnp.full((L,), cnt, jnp.int32)
        first = plsc.load_gather(idxbuf, [zeros16])
        pad = jnp.where(cntv > 0, first, zeros16)
        v0 = jnp.where(lane < cntv, v0, pad)
        v1 = jnp.where(lane + L < cntv, v1, pad)

        o = j * NSAMPLE
        idx_tile[pl.ds(o, L)] = v0
        idx_tile[pl.ds(o + L, L)] = v1
        gidx_tile[pl.ds(o, L)] = v0 + bN
        gidx_tile[pl.ds(o + L, L)] = v1 + bN
        dxt[pl.ds(o, L)] = plsc.load_gather(px, [v0]) - cxb
        dxt[pl.ds(o + L, L)] = plsc.load_gather(px, [v1]) - cxb
        dyt[pl.ds(o, L)] = plsc.load_gather(py, [v0]) - cyb
        dyt[pl.ds(o + L, L)] = plsc.load_gather(py, [v1]) - cyb
        dzt[pl.ds(o, L)] = plsc.load_gather(pz, [v0]) - czb
        dzt[pl.ds(o + L, L)] = plsc.load_gather(pz, [v1]) - czb
        return carry

    lax.fori_loop(0, SPW, per_centroid, jnp.int32(0))

    base = wid * SPW * NSAMPLE
    pltpu.sync_copy(idx_tile, idx_hbm.at[pl.ds(base, SPW * NSAMPLE)])
    pltpu.sync_copy(gidx_tile, gidx_hbm.at[pl.ds(base, SPW * NSAMPLE)])
    pltpu.sync_copy(dxt, dxyz_hbm.at[pl.ds(0 * ROWS_H + base, SPW * NSAMPLE)])
    pltpu.sync_copy(dyt, dxyz_hbm.at[pl.ds(1 * ROWS_H + base, SPW * NSAMPLE)])
    pltpu.sync_copy(dzt, dxyz_hbm.at[pl.ds(2 * ROWS_H + base, SPW * NSAMPLE)])

  return _ballquery


_ballquery_h = (_make_ballquery(0), _make_ballquery(1))


_GCHUNK = 128
_NCH = ROWS_H // NW // _GCHUNK


@functools.partial(
    pl.kernel,
    out_type=jax.ShapeDtypeStruct((ROWS_H, C), jnp.float32),
    mesh=_mesh,
    compiler_params=_sc_params,
    scratch_types=[
        pltpu.VMEM((_NCH, _GCHUNK), jnp.int32),
        pltpu.VMEM((_GCHUNK, C), jnp.float32),
        pltpu.VMEM((_GCHUNK, C), jnp.float32),
        pltpu.SemaphoreType.DMA,
        pltpu.SemaphoreType.DMA,
    ],
)
def _rowgather(ftab_hbm, gidx_hbm, out_hbm, iv_v, buf0, buf1, sem0, sem1):
    wid = _wid()
    pltpu.sync_copy(gidx_hbm.at[pl.ds(wid * _NCH, _NCH)], iv_v)
    obase = wid * _NCH * _GCHUNK

    pltpu.async_copy(ftab_hbm.at[iv_v.at[0]], buf0, sem0)

    def pair(jj, carry):
        j0 = jj * 2
        j1 = j0 + 1
        pltpu.async_copy(ftab_hbm.at[iv_v.at[j1]], buf1, sem1)
        pltpu.make_async_copy(ftab_hbm.at[iv_v.at[j0]], buf0, sem0).wait()
        pltpu.sync_copy(buf0, out_hbm.at[pl.ds(obase + j0 * _GCHUNK, _GCHUNK)])

        @pl.when(jj < _NCH // 2 - 1)
        def _():
            pltpu.async_copy(ftab_hbm.at[iv_v.at[j0 + 2]], buf0, sem0)

        pltpu.make_async_copy(ftab_hbm.at[iv_v.at[j1]], buf1, sem1).wait()
        pltpu.sync_copy(buf1, out_hbm.at[pl.ds(obase + j1 * _GCHUNK, _GCHUNK)])
        return carry

    lax.fori_loop(0, _NCH // 2, pair, jnp.int32(0))


_TB = 1024


def _pret_body(f_ref, w_ref, o_ref):
    o_ref[...] = jnp.dot(f_ref[...], w_ref[...],
                         preferred_element_type=jnp.float32)


def _pretransform(ftab, w1f):
    return pl.pallas_call(
        _pret_body,
        grid=(B * N // _TB,),
        in_specs=[
            pl.BlockSpec((_TB, C), lambda i: (i, 0)),
            pl.BlockSpec((C, D1), lambda i: (0, 0)),
        ],
        out_specs=pl.BlockSpec((_TB, D1), lambda i: (i, 0)),
        out_shape=jax.ShapeDtypeStruct((B * N, D1), jnp.float32),
    )(ftab, w1f)


_RB = 1024


def _mlp_body(fg_ref, dx_ref, w1x_ref, b1_ref, w2_ref, b2_ref,
              w3_ref, b3_ref, out_ref):
    hx = lax.dot_general(dx_ref[...], w1x_ref[...],
                         (((0,), (0,)), ((), ())),
                         preferred_element_type=jnp.float32)
    h1 = jnp.maximum(fg_ref[...] + hx + b1_ref[...], 0.0)
    h2 = jnp.maximum(
        jnp.dot(h1, w2_ref[...], preferred_element_type=jnp.float32)
        + b2_ref[...], 0.0)
    h3 = jnp.maximum(
        jnp.dot(h2, w3_ref[...], preferred_element_type=jnp.float32)
        + b3_ref[...], 0.0)
    out_ref[...] = jnp.max(h3.reshape(_RB // NSAMPLE, NSAMPLE, D3), axis=1)


def _mlp(fg, dxyz, w1x, b1, w2, b2, w3, b3):
    grid = ROWS_H // _RB
    return pl.pallas_call(
        _mlp_body,
        grid=(grid,),
        in_specs=[
            pl.BlockSpec((_RB, D1), lambda i: (i, 0)),
            pl.BlockSpec((3, _RB), lambda i: (0, i)),
            pl.BlockSpec((3, D1), lambda i: (0, 0)),
            pl.BlockSpec((1, D1), lambda i: (0, 0)),
            pl.BlockSpec((D1, D2_), lambda i: (0, 0)),
            pl.BlockSpec((1, D2_), lambda i: (0, 0)),
            pl.BlockSpec((D2_, D3), lambda i: (0, 0)),
            pl.BlockSpec((1, D3), lambda i: (0, 0)),
        ],
        out_specs=pl.BlockSpec((_RB // NSAMPLE, D3), lambda i: (i, 0)),
        out_shape=jax.ShapeDtypeStruct((SPQ, D3), jnp.float32),
    )(fg, dxyz, w1x, b1, w2, b2, w3, b3)


def kernel(xyz, features, inds, W1, b1, W2, b2, W3, b3):
    xyzT = jnp.transpose(xyz, (2, 0, 1)).reshape(3 * B * N)
    ftab = features.reshape(B * N, C)
    inds_f = inds.reshape(B * NPOINT)
    w1x = W1[:3]
    b1r = b1.reshape(1, D1)
    b2r = b2.reshape(1, D2_)
    b3r = b3.reshape(1, D3)

    tt = _pretransform(ftab, W1[3:])

    nx0, idx0, gidx0, dxyz0 = _ballquery_h[0](xyzT, inds_f[:SPQ])
    fg0 = _rowgather(tt, gidx0.reshape(ROWS_H // _GCHUNK, _GCHUNK))
    nx1, idx1, gidx1, dxyz1 = _ballquery_h[1](xyzT, inds_f[SPQ:])
    fg1 = _rowgather(tt, gidx1.reshape(ROWS_H // _GCHUNK, _GCHUNK))

    nf0 = _mlp(fg0, dxyz0.reshape(3, ROWS_H), w1x, b1r, W2, b2r, W3, b3r)
    nf1 = _mlp(fg1, dxyz1.reshape(3, ROWS_H), w1x, b1r, W2, b2r, W3, b3r)

    new_xyz = jnp.concatenate([nx0, nx1]).reshape(B, NPOINT, 3)
    idx = jnp.concatenate([idx0, idx1]).reshape(B, NPOINT, NSAMPLE)
    new_features = jnp.concatenate([nf0, nf1]).reshape(B, NPOINT, D3)
    return (new_xyz, new_features, inds, idx)


# speedup vs baseline: 1.1635x; 1.1635x over previous
import functools

import jax
import jax.numpy as jnp
import numpy as np
from jax import lax
from jax.experimental import pallas as pl
from jax.experimental.pallas import tpu as pltpu
from jax.experimental.pallas import tpu_sc as plsc

B, N, C = 4, 8192, 128
NPOINT, NSAMPLE = 1024, 32
D1, D2_, D3 = 128, 128, 256
R2 = float(np.float32(0.4) * np.float32(0.4))

NC, NS, L = 2, 16, 16
NW = NC * NS
SPW = (B * NPOINT) // NW
NGRP = N // L
ROWS = B * NPOINT * NSAMPLE

_mesh = plsc.VectorSubcoreMesh(core_axis_name="c", subcore_axis_name="s",
                               num_cores=NC, num_subcores=NS)
_sc_params = pltpu.CompilerParams(needs_layout_passes=False)


def _wid():
    return lax.axis_index("s") * NC + lax.axis_index("c")


def _rbf16(v):
    u = plsc.bitcast(v, jnp.int32)
    lsb = jnp.bitwise_and(lax.shift_right_logical(u, 16), 1)
    r = jnp.bitwise_and(u + 0x7FFF + lsb, jnp.int32(-65536))
    return plsc.bitcast(r, jnp.float32)


_GC = 128
_CPC = _GC // NSAMPLE
_NCHT = SPW // _CPC


@functools.partial(
    pl.kernel,
    out_type=[
        jax.ShapeDtypeStruct((3 * B * NPOINT,), jnp.float32),
        jax.ShapeDtypeStruct((ROWS,), jnp.int32),
        jax.ShapeDtypeStruct((3 * ROWS,), jnp.float32),
        jax.ShapeDtypeStruct((ROWS, C), jnp.float32),
    ],
    mesh=_mesh,
    compiler_params=_sc_params,
    scratch_types=[
        pltpu.VMEM((N,), jnp.float32),
        pltpu.VMEM((N,), jnp.float32),
        pltpu.VMEM((N,), jnp.float32),
        pltpu.VMEM((N,), jnp.float32),
        pltpu.VMEM((N,), jnp.float32),
        pltpu.VMEM((N,), jnp.float32),
        pltpu.VMEM((N,), jnp.float32),
        pltpu.VMEM((SPW,), jnp.int32),
        pltpu.VMEM((SPW,), jnp.float32),
        pltpu.VMEM((SPW,), jnp.float32),
        pltpu.VMEM((SPW,), jnp.float32),
        pltpu.VMEM((48,), jnp.int32),
        pltpu.VMEM((SPW * NSAMPLE,), jnp.int32),
        pltpu.VMEM((_NCHT, _GC), jnp.int32),
        pltpu.VMEM((SPW * NSAMPLE,), jnp.float32),
        pltpu.VMEM((SPW * NSAMPLE,), jnp.float32),
        pltpu.VMEM((SPW * NSAMPLE,), jnp.float32),
        pltpu.VMEM((_GC, C), jnp.float32),
        pltpu.VMEM((_GC, C), jnp.float32),
        pltpu.SemaphoreType.DMA,
        pltpu.SemaphoreType.DMA,
        pltpu.SemaphoreType.DMA,
        pltpu.SemaphoreType.DMA,
    ],
)
def _ballquery(xyzT_hbm, inds_hbm, ttab_hbm, nx_hbm, idx_hbm, dxyz_hbm,
               fg_hbm, px, py, pz, pxr, pyr, pzr, x2a, ind_v, cxc, cyc, czc,
               idxbuf, idx_tile, gidx2, dxt, dyt, dzt, buf0, buf1,
               g0, g1, w0, w1):
    wid = _wid()
    wpb = NW // B
    b = wid // wpb
    srel = (wid % wpb) * SPW

    pltpu.sync_copy(xyzT_hbm.at[pl.ds(0 * B * N + b * N, N)], px)
    pltpu.sync_copy(xyzT_hbm.at[pl.ds(1 * B * N + b * N, N)], py)
    pltpu.sync_copy(xyzT_hbm.at[pl.ds(2 * B * N + b * N, N)], pz)
    pltpu.sync_copy(inds_hbm.at[pl.ds(wid * SPW, SPW)], ind_v)

    def round_grp(g, carry):
        xs = px[pl.ds(g * L, L)]
        ys = py[pl.ds(g * L, L)]
        zs = pz[pl.ds(g * L, L)]
        pxr[pl.ds(g * L, L)] = _rbf16(xs)
        pyr[pl.ds(g * L, L)] = _rbf16(ys)
        pzr[pl.ds(g * L, L)] = _rbf16(zs)
        x2a[pl.ds(g * L, L)] = (xs * xs + ys * ys) + zs * zs
        return carry

    lax.fori_loop(0, NGRP, round_grp, jnp.int32(0))

    for g in range(SPW // L):
        iv = ind_v[pl.ds(g * L, L)]
        cxc[pl.ds(g * L, L)] = plsc.load_gather(px, [iv])
        cyc[pl.ds(g * L, L)] = plsc.load_gather(py, [iv])
        czc[pl.ds(g * L, L)] = plsc.load_gather(pz, [iv])
    nq = B * NPOINT
    pltpu.sync_copy(cxc, nx_hbm.at[pl.ds(0 * nq + wid * SPW, SPW)])
    pltpu.sync_copy(cyc, nx_hbm.at[pl.ds(1 * nq + wid * SPW, SPW)])
    pltpu.sync_copy(czc, nx_hbm.at[pl.ds(2 * nq + wid * SPW, SPW)])

    lane = lax.iota(jnp.int32, L)
    zeros16 = jnp.zeros((L,), jnp.int32)
    ones16 = jnp.ones((L,), jnp.int32)
    bN = b * N
    rbase = wid * SPW * NSAMPLE

    def centroid(j, c, k):
        jv = jnp.full((L,), j, jnp.int32)
        cxb = plsc.load_gather(cxc, [jv])
        cyb = plsc.load_gather(cyc, [jv])
        czb = plsc.load_gather(czc, [jv])
        q2 = (cxb * cxb + cyb * cyb) + czb * czb
        cxh, cyh, czh = _rbf16(cxb), _rbf16(cyb), _rbf16(czb)

        def cond(st):
            g, cnt = st
            return jnp.logical_and(g < NGRP, cnt < NSAMPLE)

        def body(st):
            g, cnt = st
            dot = (cxh * pxr[pl.ds(g * L, L)] + cyh * pyr[pl.ds(g * L, L)]) \
                + czh * pzr[pl.ds(g * L, L)]
            d2 = (q2 + x2a[pl.ds(g * L, L)]) - 2.0 * dot
            m = d2 < R2
            plsc.store_compressed(idxbuf.at[pl.ds(cnt, L)], lane + g * L, mask=m)
            cnt = cnt + jnp.sum(jnp.where(m, ones16, zeros16))
            return g + 1, cnt

        _, cnt = lax.while_loop(cond, body,
                                (jnp.int32(0), jnp.int32(0)))

        v0 = idxbuf[pl.ds(0, L)]
        v1 = idxbuf[pl.ds(L, L)]
        cntv = jnp.full((L,), cnt, jnp.int32)
        first = plsc.load_gather(idxbuf, [zeros16])
        pad = jnp.where(cntv > 0, first, zeros16)
        v0 = jnp.where(lane < cntv, v0, pad)
        v1 = jnp.where(lane + L < cntv, v1, pad)

        o = j * NSAMPLE
        idx_tile[pl.ds(o, L)] = v0
        idx_tile[pl.ds(o + L, L)] = v1
        gidx2[c, pl.ds(k * NSAMPLE, L)] = v0 + bN
        gidx2[c, pl.ds(k * NSAMPLE + L, L)] = v1 + bN
        dxt[pl.ds(o, L)] = plsc.load_gather(px, [v0]) - cxb
        dxt[pl.ds(o + L, L)] = plsc.load_gather(px, [v1]) - cxb
        dyt[pl.ds(o, L)] = plsc.load_gather(py, [v0]) - cyb
        dyt[pl.ds(o + L, L)] = plsc.load_gather(py, [v1]) - cyb
        dzt[pl.ds(o, L)] = plsc.load_gather(pz, [v0]) - czb
        dzt[pl.ds(o + L, L)] = plsc.load_gather(pz, [v1]) - czb

    bufs = (buf0, buf1)
    gsems = (g0, g1)
    wsems = (w0, w1)

    def gref(c):
        return ttab_hbm.at[gidx2.at[c]]

    def oref(c):
        return fg_hbm.at[pl.ds(rbase + c * _GC, _GC)]

    def super_chunk(cc, carry):
        for s in range(2):
            c = cc * 2 + s
            p = 1 - s
            for k in range(_CPC):
                centroid(c * _CPC + k, c, k)

            def wb_prev():
                pltpu.make_async_copy(gref(c - 1), bufs[p], gsems[p]).wait()
                pltpu.async_copy(bufs[p], oref(c - 1), wsems[p])

            if s == 1:
                wb_prev()
            else:
                @pl.when(cc > 0)
                def _():
                    wb_prev()

            @pl.when(cc > 0)
            def _():
                pltpu.make_async_copy(bufs[s], oref(c - 2), wsems[s]).wait()

            pltpu.async_copy(gref(c), bufs[s], gsems[s])
        return carry

    lax.fori_loop(0, _NCHT // 2, super_chunk, jnp.int32(0))

    last = _NCHT - 1
    pltpu.make_async_copy(gref(last), buf1, g1).wait()
    pltpu.sync_copy(buf1, oref(last))
    pltpu.make_async_copy(buf0, oref(last - 1), w0).wait()

    base = wid * SPW * NSAMPLE
    pltpu.sync_copy(idx_tile, idx_hbm.at[pl.ds(base, SPW * NSAMPLE)])
    pltpu.sync_copy(dxt, dxyz_hbm.at[pl.ds(0 * ROWS + base, SPW * NSAMPLE)])
    pltpu.sync_copy(dyt, dxyz_hbm.at[pl.ds(1 * ROWS + base, SPW * NSAMPLE)])
    pltpu.sync_copy(dzt, dxyz_hbm.at[pl.ds(2 * ROWS + base, SPW * NSAMPLE)])


_TB = 1024


def _pret_body(f_ref, w_ref, o_ref):
    o_ref[...] = jnp.dot(f_ref[...], w_ref[...],
                         preferred_element_type=jnp.float32)


def _pretransform(ftab, w1f):
    return pl.pallas_call(
        _pret_body,
        grid=(B * N // _TB,),
        in_specs=[
            pl.BlockSpec((_TB, C), lambda i: (i, 0)),
            pl.BlockSpec((C, D1), lambda i: (0, 0)),
        ],
        out_specs=pl.BlockSpec((_TB, D1), lambda i: (i, 0)),
        out_shape=jax.ShapeDtypeStruct((B * N, D1), jnp.float32),
    )(ftab, w1f)


_RB = 1024


def _mlp_body(fg_ref, dx_ref, w1x_ref, b1_ref, w2_ref, b2_ref,
              w3_ref, b3_ref, out_ref):
    hx = lax.dot_general(dx_ref[...], w1x_ref[...],
                         (((0,), (0,)), ((), ())),
                         preferred_element_type=jnp.float32)
    h1 = jnp.maximum(fg_ref[...] + hx + b1_ref[...], 0.0)
    h2 = jnp.maximum(
        jnp.dot(h1, w2_ref[...], preferred_element_type=jnp.float32)
        + b2_ref[...], 0.0)
    h3 = jnp.maximum(
        jnp.dot(h2, w3_ref[...], preferred_element_type=jnp.float32)
        + b3_ref[...], 0.0)
    out_ref[...] = jnp.max(h3.reshape(_RB // NSAMPLE, NSAMPLE, D3), axis=1)


def _mlp(fg, dxyz, w1x, b1, w2, b2, w3, b3):
    grid = ROWS // _RB
    return pl.pallas_call(
        _mlp_body,
        grid=(grid,),
        in_specs=[
            pl.BlockSpec((_RB, D1), lambda i: (i, 0)),
            pl.BlockSpec((3, _RB), lambda i: (0, i)),
            pl.BlockSpec((3, D1), lambda i: (0, 0)),
            pl.BlockSpec((1, D1), lambda i: (0, 0)),
            pl.BlockSpec((D1, D2_), lambda i: (0, 0)),
            pl.BlockSpec((1, D2_), lambda i: (0, 0)),
            pl.BlockSpec((D2_, D3), lambda i: (0, 0)),
            pl.BlockSpec((1, D3), lambda i: (0, 0)),
        ],
        out_specs=pl.BlockSpec((_RB // NSAMPLE, D3), lambda i: (i, 0)),
        out_shape=jax.ShapeDtypeStruct((B * NPOINT, D3), jnp.float32),
    )(fg, dxyz, w1x, b1, w2, b2, w3, b3)


def kernel(xyz, features, inds, W1, b1, W2, b2, W3, b3):
    xyzT = jnp.transpose(xyz, (2, 0, 1)).reshape(3 * B * N)
    ftab = features.reshape(B * N, C)

    tt = _pretransform(ftab, W1[3:])
    nxT, idx_flat, dxyz, fg = _ballquery(xyzT, inds.reshape(B * NPOINT), tt)

    nf = _mlp(fg, dxyz.reshape(3, ROWS), W1[:3], b1.reshape(1, D1),
              W2, b2.reshape(1, D2_), W3, b3.reshape(1, D3))

    new_xyz = nxT.reshape(3, B * NPOINT).T.reshape(B, NPOINT, 3)
    idx = idx_flat.reshape(B, NPOINT, NSAMPLE)
    new_features = nf.reshape(B, NPOINT, D3)
    return (new_xyz, new_features, inds, idx)


# speedup vs baseline: 1.2548x; 1.0785x over previous
import functools

import jax
import jax.numpy as jnp
import numpy as np
from jax import lax
from jax.experimental import pallas as pl
from jax.experimental.pallas import tpu as pltpu
from jax.experimental.pallas import tpu_sc as plsc

B, N, C = 4, 8192, 128
NPOINT, NSAMPLE = 1024, 32
D1, D2_, D3 = 128, 128, 256
R2 = float(np.float32(0.4) * np.float32(0.4))

NC, NS, L = 2, 16, 16
NW = NC * NS
SPW = (B * NPOINT) // NW
NGRP = N // L
ROWS = B * NPOINT * NSAMPLE

_mesh = plsc.VectorSubcoreMesh(core_axis_name="c", subcore_axis_name="s",
                               num_cores=NC, num_subcores=NS)
_sc_params = pltpu.CompilerParams(needs_layout_passes=False)


def _wid():
    return lax.axis_index("s") * NC + lax.axis_index("c")


def _rbf16(v):
    u = plsc.bitcast(v, jnp.int32)
    lsb = jnp.bitwise_and(lax.shift_right_logical(u, 16), 1)
    r = jnp.bitwise_and(u + 0x7FFF + lsb, jnp.int32(-65536))
    return plsc.bitcast(r, jnp.float32)


_GC = 128
_CPC = _GC // NSAMPLE
_NCHT = SPW // _CPC


@functools.partial(
    pl.kernel,
    out_type=[
        jax.ShapeDtypeStruct((B * NPOINT * 3,), jnp.float32),
        jax.ShapeDtypeStruct((ROWS,), jnp.int32),
        jax.ShapeDtypeStruct((3 * ROWS,), jnp.float32),
        jax.ShapeDtypeStruct((ROWS, C), jnp.float32),
    ],
    mesh=_mesh,
    compiler_params=_sc_params,
    scratch_types=[
        pltpu.VMEM((N * 3,), jnp.float32),
        pltpu.VMEM((N,), jnp.float32),
        pltpu.VMEM((N,), jnp.float32),
        pltpu.VMEM((N,), jnp.float32),
        pltpu.VMEM((N,), jnp.float32),
        pltpu.VMEM((SPW,), jnp.int32),
        pltpu.VMEM((SPW,), jnp.float32),
        pltpu.VMEM((SPW,), jnp.float32),
        pltpu.VMEM((SPW,), jnp.float32),
        pltpu.VMEM((SPW * 3,), jnp.float32),
        pltpu.VMEM((48,), jnp.int32),
        pltpu.VMEM((SPW * NSAMPLE,), jnp.int32),
        pltpu.VMEM((_NCHT, _GC), jnp.int32),
        pltpu.VMEM((SPW * NSAMPLE,), jnp.float32),
        pltpu.VMEM((SPW * NSAMPLE,), jnp.float32),
        pltpu.VMEM((SPW * NSAMPLE,), jnp.float32),
        pltpu.VMEM((_GC, C), jnp.float32),
        pltpu.VMEM((_GC, C), jnp.float32),
        pltpu.SemaphoreType.DMA,
        pltpu.SemaphoreType.DMA,
        pltpu.SemaphoreType.DMA,
        pltpu.SemaphoreType.DMA,
    ],
)
def _ballquery(xyz_hbm, inds_hbm, ttab_hbm, nx_hbm, idx_hbm, dxyz_hbm,
               fg_hbm, xi, pxr, pyr, pzr, x2a, ind_v, cxc, cyc, czc,
               nx_tile, idxbuf, idx_tile, gidx2, dxt, dyt, dzt, buf0, buf1,
               g0, g1, w0, w1):
    wid = _wid()
    wpb = NW // B
    b = wid // wpb

    pltpu.sync_copy(xyz_hbm.at[pl.ds(b * N * 3, N * 3)], xi)
    pltpu.sync_copy(inds_hbm.at[pl.ds(wid * SPW, SPW)], ind_v)

    lane = lax.iota(jnp.int32, L)

    def round_grp(g, carry):
        i3 = (lane + g * L) * 3
        xs = plsc.load_gather(xi, [i3])
        ys = plsc.load_gather(xi, [i3 + 1])
        zs = plsc.load_gather(xi, [i3 + 2])
        pxr[pl.ds(g * L, L)] = _rbf16(xs)
        pyr[pl.ds(g * L, L)] = _rbf16(ys)
        pzr[pl.ds(g * L, L)] = _rbf16(zs)
        x2a[pl.ds(g * L, L)] = (xs * xs + ys * ys) + zs * zs
        return carry

    lax.fori_loop(0, NGRP, round_grp, jnp.int32(0))

    for g in range(SPW // L):
        iv3 = ind_v[pl.ds(g * L, L)] * 3
        cxg = plsc.load_gather(xi, [iv3])
        cyg = plsc.load_gather(xi, [iv3 + 1])
        czg = plsc.load_gather(xi, [iv3 + 2])
        cxc[pl.ds(g * L, L)] = cxg
        cyc[pl.ds(g * L, L)] = cyg
        czc[pl.ds(g * L, L)] = czg
        tri = 3 * (lane + g * L)
        plsc.store_scatter(nx_tile, [tri], cxg)
        plsc.store_scatter(nx_tile, [tri + 1], cyg)
        plsc.store_scatter(nx_tile, [tri + 2], czg)
    pltpu.sync_copy(nx_tile, nx_hbm.at[pl.ds(wid * SPW * 3, SPW * 3)])
    zeros16 = jnp.zeros((L,), jnp.int32)
    ones16 = jnp.ones((L,), jnp.int32)
    bN = b * N
    rbase = wid * SPW * NSAMPLE

    def centroid(j, c, k):
        jv = jnp.full((L,), j, jnp.int32)
        cxb = plsc.load_gather(cxc, [jv])
        cyb = plsc.load_gather(cyc, [jv])
        czb = plsc.load_gather(czc, [jv])
        q2 = (cxb * cxb + cyb * cyb) + czb * czb
        cxh, cyh, czh = _rbf16(cxb), _rbf16(cyb), _rbf16(czb)

        def cond(st):
            g, cnt = st
            return jnp.logical_and(g < NGRP, cnt < NSAMPLE)

        def body(st):
            g, cnt = st
            dot = (cxh * pxr[pl.ds(g * L, L)] + cyh * pyr[pl.ds(g * L, L)]) \
                + czh * pzr[pl.ds(g * L, L)]
            d2 = (q2 + x2a[pl.ds(g * L, L)]) - 2.0 * dot
            m = d2 < R2
            plsc.store_compressed(idxbuf.at[pl.ds(cnt, L)], lane + g * L, mask=m)
            cnt = cnt + jnp.sum(jnp.where(m, ones16, zeros16))
            return g + 1, cnt

        _, cnt = lax.while_loop(cond, body,
                                (jnp.int32(0), jnp.int32(0)))

        v0 = idxbuf[pl.ds(0, L)]
        v1 = idxbuf[pl.ds(L, L)]
        cntv = jnp.full((L,), cnt, jnp.int32)
        first = plsc.load_gather(idxbuf, [zeros16])
        pad = jnp.where(cntv > 0, first, zeros16)
        v0 = jnp.where(lane < cntv, v0, pad)
        v1 = jnp.where(lane + L < cntv, v1, pad)

        o = j * NSAMPLE
        idx_tile[pl.ds(o, L)] = v0
        idx_tile[pl.ds(o + L, L)] = v1
        gidx2[c, pl.ds(k * NSAMPLE, L)] = v0 + bN
        gidx2[c, pl.ds(k * NSAMPLE + L, L)] = v1 + bN
        v03, v13 = v0 * 3, v1 * 3
        dxt[pl.ds(o, L)] = plsc.load_gather(xi, [v03]) - cxb
        dxt[pl.ds(o + L, L)] = plsc.load_gather(xi, [v13]) - cxb
        dyt[pl.ds(o, L)] = plsc.load_gather(xi, [v03 + 1]) - cyb
        dyt[pl.ds(o + L, L)] = plsc.load_gather(xi, [v13 + 1]) - cyb
        dzt[pl.ds(o, L)] = plsc.load_gather(xi, [v03 + 2]) - czb
        dzt[pl.ds(o + L, L)] = plsc.load_gather(xi, [v13 + 2]) - czb

    bufs = (buf0, buf1)
    gsems = (g0, g1)
    wsems = (w0, w1)

    def gref(c):
        return ttab_hbm.at[gidx2.at[c]]

    def oref(c):
        return fg_hbm.at[pl.ds(rbase + c * _GC, _GC)]

    def super_chunk(cc, carry):
        for s in range(2):
            c = cc * 2 + s
            p = 1 - s
            for k in range(_CPC):
                centroid(c * _CPC + k, c, k)

            def wb_prev():
                pltpu.make_async_copy(gref(c - 1), bufs[p], gsems[p]).wait()
                pltpu.async_copy(bufs[p], oref(c - 1), wsems[p])

            if s == 1:
                wb_prev()
            else:
                @pl.when(cc > 0)
                def _():
                    wb_prev()

            @pl.when(cc > 0)
            def _():
                pltpu.make_async_copy(bufs[s], oref(c - 2), wsems[s]).wait()

            pltpu.async_copy(gref(c), bufs[s], gsems[s])
        return carry

    lax.fori_loop(0, _NCHT // 2, super_chunk, jnp.int32(0))

    last = _NCHT - 1
    pltpu.make_async_copy(gref(last), buf1, g1).wait()
    pltpu.sync_copy(buf1, oref(last))
    pltpu.make_async_copy(buf0, oref(last - 1), w0).wait()

    base = wid * SPW * NSAMPLE
    pltpu.sync_copy(idx_tile, idx_hbm.at[pl.ds(base, SPW * NSAMPLE)])
    pltpu.sync_copy(dxt, dxyz_hbm.at[pl.ds(0 * ROWS + base, SPW * NSAMPLE)])
    pltpu.sync_copy(dyt, dxyz_hbm.at[pl.ds(1 * ROWS + base, SPW * NSAMPLE)])
    pltpu.sync_copy(dzt, dxyz_hbm.at[pl.ds(2 * ROWS + base, SPW * NSAMPLE)])


_TB = 1024


def _pret_body(f_ref, w_ref, o_ref):
    o_ref[...] = jnp.dot(f_ref[...], w_ref[...],
                         preferred_element_type=jnp.float32)


def _pretransform(ftab, w1f):
    return pl.pallas_call(
        _pret_body,
        grid=(B * N // _TB,),
        in_specs=[
            pl.BlockSpec((_TB, C), lambda i: (i, 0)),
            pl.BlockSpec((C, D1), lambda i: (0, 0)),
        ],
        out_specs=pl.BlockSpec((_TB, D1), lambda i: (i, 0)),
        out_shape=jax.ShapeDtypeStruct((B * N, D1), jnp.float32),
    )(ftab, w1f)


_RB = 2048


def _mlp_body(fg_ref, dx_ref, w1x_ref, b1_ref, w2_ref, b2_ref,
              w3_ref, b3_ref, out_ref):
    hx = lax.dot_general(dx_ref[...], w1x_ref[...],
                         (((0,), (0,)), ((), ())),
                         preferred_element_type=jnp.float32)
    h1 = jnp.maximum(fg_ref[...] + hx + b1_ref[...], 0.0)
    h2 = jnp.maximum(
        jnp.dot(h1, w2_ref[...], preferred_element_type=jnp.float32)
        + b2_ref[...], 0.0)
    h3 = jnp.maximum(
        jnp.dot(h2, w3_ref[...], preferred_element_type=jnp.float32)
        + b3_ref[...], 0.0)
    out_ref[...] = jnp.max(h3.reshape(_RB // NSAMPLE, NSAMPLE, D3), axis=1)


def _mlp(fg, dxyz, w1x, b1, w2, b2, w3, b3):
    grid = ROWS // _RB
    return pl.pallas_call(
        _mlp_body,
        grid=(grid,),
        in_specs=[
            pl.BlockSpec((_RB, D1), lambda i: (i, 0)),
            pl.BlockSpec((3, _RB), lambda i: (0, i)),
            pl.BlockSpec((3, D1), lambda i: (0, 0)),
            pl.BlockSpec((1, D1), lambda i: (0, 0)),
            pl.BlockSpec((D1, D2_), lambda i: (0, 0)),
            pl.BlockSpec((1, D2_), lambda i: (0, 0)),
            pl.BlockSpec((D2_, D3), lambda i: (0, 0)),
            pl.BlockSpec((1, D3), lambda i: (0, 0)),
        ],
        out_specs=pl.BlockSpec((_RB // NSAMPLE, D3), lambda i: (i, 0)),
        out_shape=jax.ShapeDtypeStruct((B * NPOINT, D3), jnp.float32),
    )(fg, dxyz, w1x, b1, w2, b2, w3, b3)


def kernel(xyz, features, inds, W1, b1, W2, b2, W3, b3):
    ftab = features.reshape(B * N, C)

    tt = _pretransform(ftab, W1[3:])
    nx, idx_flat, dxyz, fg = _ballquery(xyz.reshape(B * N * 3),
                                        inds.reshape(B * NPOINT), tt)

    nf = _mlp(fg, dxyz.reshape(3, ROWS), W1[:3], b1.reshape(1, D1),
              W2, b2.reshape(1, D2_), W3, b3.reshape(1, D3))

    new_xyz = nx.reshape(B, NPOINT, 3)
    idx = idx_flat.reshape(B, NPOINT, NSAMPLE)
    new_features = nf.reshape(B, NPOINT, D3)
    return (new_xyz, new_features, inds, idx)
